# baseline (speedup 1.0000x reference)
import functools

import numpy as np
import jax
import jax.numpy as jnp
from jax import lax
from jax.experimental import pallas as pl
from jax.experimental.pallas import tpu as pltpu
from jax.experimental.pallas import tpu_sc as plsc

N_NODES_STATIC = 10000
N_EDGES = 320000
EDGE_DIM = 128
N_RADIAL = 16

NC = 2
NS = 16
NW = NC * NS

GROUP = 128
N_CHUNKS = 4
CHUNK_EDGES = N_EDGES // N_CHUNKS
CHUNK_GROUPS = CHUNK_EDGES // GROUP
RPW = CHUNK_GROUPS // NW
N_TAIL = CHUNK_GROUPS - NW * RPW
ZROWS = 624
ZTAIL = N_NODES_STATIC - NS * ZROWS

BE = 16000
HB = BE // 2
GPB = HB // 64


def _gate_body(rbf_ref, x_ref, w_ref, h_ref):
    g = jnp.dot(rbf_ref[...], w_ref[...], preferred_element_type=jnp.float32)
    u = lax.bitcast_convert_type(g * x_ref[...], jnp.int32)
    rb = ((u + 0x7FFF + ((u >> 16) & 1)) >> 16) & 0xFFFF
    h_ref[...] = rb[:HB, :] | (rb[HB:, :] << 16)


def _gating(rbf, x, W_rbf, chunk):
    nb = CHUNK_EDGES // BE
    base = chunk * nb
    return pl.pallas_call(
        _gate_body,
        grid=(nb,),
        in_specs=[
            pl.BlockSpec((BE, N_RADIAL), lambda i: (base + i, 0)),
            pl.BlockSpec((BE, EDGE_DIM), lambda i: (base + i, 0)),
            pl.BlockSpec((N_RADIAL, EDGE_DIM), lambda i: (0, 0)),
        ],
        out_specs=pl.BlockSpec((HB, EDGE_DIM), lambda i: (i, 0)),
        out_shape=jax.ShapeDtypeStruct((CHUNK_EDGES // 2, EDGE_DIM), jnp.int32),
    )(rbf, x, W_rbf)


def _sc_scatter_body(h_hbm, idx_hbm, out_hbm, idx_v, idx_sc, rows_bf0,
                     rows_bf1, rows_f0, rows_f1, acc, sem0, sem1, ssem0,
                     ssem1):
    c = lax.axis_index("c")
    s = lax.axis_index("s")
    w = s * NC + c

    z16 = jnp.zeros((16,), jnp.float32)

    def zero_body(t, carry):
        rows_f0[t // 8, pl.ds((t % 8) * 16, 16)] = z16
        return carry

    lax.fori_loop(0, GROUP * 8, zero_body, 0)
    for zi, zn in ((0, 128), (128, 128), (256, 128), (384, 128), (512, 112)):
        pltpu.sync_copy(
            rows_f0.at[pl.ds(0, zn)],
            acc.at[pl.ds(s * ZROWS + zi, zn)],
        )

    @pl.when(s == NS - 1)
    def _zero_tail():
        pltpu.sync_copy(
            rows_f0.at[pl.ds(0, ZTAIL)],
            acc.at[pl.ds(NS * ZROWS, ZTAIL)],
        )

    plsc.subcore_barrier()

    g0 = w * RPW
    sems = (sem0, sem1)

    GR = GROUP // 2
    bufs = (rows_bf0, rows_bf1)
    rows_fs = (rows_f0, rows_f1)
    ssems = (ssem0, ssem1)
    _himask = jnp.full((16,), -65536, jnp.int32)

    def start(g, b):
        sem = sems[b]
        e0 = (g // GPB) * BE + (g % GPB) * 64
        pltpu.async_copy(h_hbm.at[pl.ds(g * GR, GR)], bufs[b], sem)
        pltpu.async_copy(idx_hbm.at[pl.ds(e0, 64)],
                         idx_v.at[b, pl.ds(0, 64)], sem)
        pltpu.async_copy(idx_hbm.at[pl.ds(e0 + HB, 64)],
                         idx_v.at[b, pl.ds(64, 64)], sem)

    def convert(b):
        def cv_body(r, carry):
            for cb in range(EDGE_DIM // 16):
                co = cb * 16
                v = bufs[b][r, pl.ds(co, 16)]
                rows_fs[b][r, pl.ds(co, 16)] = lax.bitcast_convert_type(
                    v << 16, jnp.float32)
                rows_fs[b][GR + r, pl.ds(co, 16)] = lax.bitcast_convert_type(
                    v & _himask, jnp.float32)
            return carry

        lax.fori_loop(0, GR, cv_body, 0)
        for k in range(GROUP // 16):
            idx_sc[b, pl.ds(k * 16, 16)] = idx_v[b, pl.ds(k * 16, 16)]

    def wait_dma(b):
        sem = sems[b]
        pltpu.make_async_copy(
            h_hbm.at[pl.ds(0, GR)], bufs[b], sem).wait()
        pltpu.make_async_copy(
            idx_hbm.at[pl.ds(0, 64)], idx_v.at[b, pl.ds(0, 64)], sem).wait()
        pltpu.make_async_copy(
            idx_hbm.at[pl.ds(0, 64)], idx_v.at[b, pl.ds(64, 64)], sem).wait()

    def start_scatter(b):
        pltpu.async_copy(rows_fs[b], acc.at[idx_sc.at[b]], ssems[b], add=True)

    def wait_scatter(b):
        pltpu.make_async_copy(
            rows_fs[b], acc.at[idx_sc.at[b]], ssems[b]).wait()

    start(g0, 0)

    def group_body(t, carry):
        start(g0 + 2 * t + 1, 1)
        wait_dma(0)

        @pl.when(t > 0)
        def _drain0():
            wait_scatter(0)

        convert(0)
        start_scatter(0)

        @pl.when(2 * t + 2 < RPW)
        def _start_next():
            start(g0 + 2 * t + 2, 0)

        wait_dma(1)

        @pl.when(t > 0)
        def _drain1():
            wait_scatter(1)

        convert(1)
        start_scatter(1)
        return carry

    lax.fori_loop(0, RPW // 2, group_body, 0)
    if RPW % 2 == 1:
        wait_dma(0)
        wait_scatter(0)
        convert(0)
        start_scatter(0)
    wait_scatter(1)
    wait_scatter(0)

    @pl.when(w < N_TAIL)
    def _tail():
        gt = NW * RPW + w
        et = (gt // GPB) * BE + (gt % GPB) * 64
        pltpu.sync_copy(h_hbm.at[pl.ds(gt * GR, GR)], bufs[0])
        pltpu.sync_copy(idx_hbm.at[pl.ds(et, 64)], idx_v.at[0, pl.ds(0, 64)])
        pltpu.sync_copy(idx_hbm.at[pl.ds(et + HB, 64)],
                        idx_v.at[0, pl.ds(64, 64)])
        convert(0)
        pltpu.sync_copy(rows_fs[0], acc.at[idx_sc.at[0]], add=True)

    plsc.subcore_barrier()
    pltpu.sync_copy(
        acc.at[pl.ds(s * ZROWS, ZROWS)],
        out_hbm.at[c, pl.ds(s * ZROWS, ZROWS)],
    )

    @pl.when(s == NS - 1)
    def _write_tail():
        pltpu.sync_copy(
            acc.at[pl.ds(NS * ZROWS, ZTAIL)],
            out_hbm.at[c, pl.ds(NS * ZROWS, ZTAIL)],
        )


def _sc_scatter(h, idx1d):
    mesh = plsc.VectorSubcoreMesh(core_axis_name="c", subcore_axis_name="s")
    kfn = pl.kernel(
        _sc_scatter_body,
        out_type=jax.ShapeDtypeStruct((NC, N_NODES_STATIC, EDGE_DIM), jnp.float32),
        mesh=mesh,
        scratch_types=[
            pltpu.VMEM((2, GROUP), jnp.int32),
            pltpu.VMEM((2, GROUP), jnp.int32),
            pltpu.VMEM((GROUP // 2, EDGE_DIM), jnp.int32),
            pltpu.VMEM((GROUP // 2, EDGE_DIM), jnp.int32),
            pltpu.VMEM((GROUP, EDGE_DIM), jnp.float32),
            pltpu.VMEM((GROUP, EDGE_DIM), jnp.float32),
            pltpu.VMEM_SHARED((N_NODES_STATIC, EDGE_DIM), jnp.float32),
            pltpu.SemaphoreType.DMA,
            pltpu.SemaphoreType.DMA,
            pltpu.SemaphoreType.DMA,
            pltpu.SemaphoreType.DMA,
        ],
    )
    return kfn(h, idx1d)


def _mlp_body(p0_ref, p1_ref, p2_ref, p3_ref, w1, b1, w2, b2, w3, b3, wo,
              o_ref):
    a = ((p0_ref[0] + p0_ref[1]) + (p1_ref[0] + p1_ref[1])
         + (p2_ref[0] + p2_ref[1]) + (p3_ref[0] + p3_ref[1]))
    for wref, bref in ((w1, b1), (w2, b2), (w3, b3)):
        a = jnp.dot(a, wref[...], preferred_element_type=jnp.float32) + bref[...]
        a = a * (1.0 / (1.0 + jnp.exp(-a)))
    o_ref[...] = jnp.dot(a, wo[...], preferred_element_type=jnp.float32)


def _mlp(p0, p1, p2, p3, W1, b1, W2, b2, W3, b3, W_out):
    BN = 2000
    grid = (N_NODES_STATIC // BN,)
    full = lambda shape: pl.BlockSpec(shape, lambda i: tuple(0 for _ in shape))
    return pl.pallas_call(
        _mlp_body,
        grid=grid,
        in_specs=[
            pl.BlockSpec((NC, BN, EDGE_DIM), lambda i: (0, i, 0)),
            pl.BlockSpec((NC, BN, EDGE_DIM), lambda i: (0, i, 0)),
            pl.BlockSpec((NC, BN, EDGE_DIM), lambda i: (0, i, 0)),
            pl.BlockSpec((NC, BN, EDGE_DIM), lambda i: (0, i, 0)),
            full((EDGE_DIM, EDGE_DIM)),
            full((1, EDGE_DIM)),
            full((EDGE_DIM, EDGE_DIM)),
            full((1, EDGE_DIM)),
            full((EDGE_DIM, EDGE_DIM)),
            full((1, EDGE_DIM)),
            full((EDGE_DIM, 1)),
        ],
        out_specs=pl.BlockSpec((BN, 1), lambda i: (i, 0)),
        out_shape=jax.ShapeDtypeStruct((N_NODES_STATIC, 1), jnp.float32),
    )(p0, p1, p2, p3, W1, b1.reshape(1, -1), W2, b2.reshape(1, -1),
      W3, b3.reshape(1, -1), W_out)


def kernel(x, rbf, idx_i, num_nodes, W_rbf, W1, b1, W2, b2, W3, b3, W_out):
    idx = idx_i.astype(jnp.int32) + (
        jnp.asarray(num_nodes, jnp.int32) - N_NODES_STATIC)

    partials = []
    for chunk in range(N_CHUNKS):
        h32 = _gating(rbf, x, W_rbf, chunk)
        idx_c = lax.slice(idx, (chunk * CHUNK_EDGES,),
                          ((chunk + 1) * CHUNK_EDGES,))
        partials.append(_sc_scatter(h32, idx_c))
    return _mlp(*partials, W1, b1, W2, b2, W3, b3, W_out)


# speedup vs baseline: 1.0797x; 1.0797x over previous
import functools

import numpy as np
import jax
import jax.numpy as jnp
from jax import lax
from jax.experimental import pallas as pl
from jax.experimental.pallas import tpu as pltpu
from jax.experimental.pallas import tpu_sc as plsc

N_NODES_STATIC = 10000
N_EDGES = 320000
EDGE_DIM = 128
N_RADIAL = 16

NC = 2
NS = 16
NW = NC * NS

GROUP = 128
N_CHUNKS = 2
CHUNK_EDGES = N_EDGES // N_CHUNKS
CHUNK_GROUPS = CHUNK_EDGES // GROUP
RPW = CHUNK_GROUPS // NW
N_TAIL = CHUNK_GROUPS - NW * RPW
ZROWS = 624
ZTAIL = N_NODES_STATIC - NS * ZROWS

BE = 16000
HB = BE // 2
GPB = HB // 64


def _gate_body(rbf_ref, x_ref, w_ref, h_ref):
    g = jnp.dot(rbf_ref[...], w_ref[...], preferred_element_type=jnp.float32)
    u = lax.bitcast_convert_type(g * x_ref[...], jnp.int32)
    rb = ((u + 0x7FFF + ((u >> 16) & 1)) >> 16) & 0xFFFF
    h_ref[...] = rb[:HB, :] | (rb[HB:, :] << 16)


def _gating(rbf, x, W_rbf, chunk):
    nb = CHUNK_EDGES // BE
    base = chunk * nb
    return pl.pallas_call(
        _gate_body,
        grid=(nb,),
        in_specs=[
            pl.BlockSpec((BE, N_RADIAL), lambda i: (base + i, 0)),
            pl.BlockSpec((BE, EDGE_DIM), lambda i: (base + i, 0)),
            pl.BlockSpec((N_RADIAL, EDGE_DIM), lambda i: (0, 0)),
        ],
        out_specs=pl.BlockSpec((HB, EDGE_DIM), lambda i: (i, 0)),
        out_shape=jax.ShapeDtypeStruct((CHUNK_EDGES // 2, EDGE_DIM), jnp.int32),
    )(rbf, x, W_rbf)


def _sc_scatter_body(h_hbm, idx_hbm, out_hbm, idx_v, idx_sc, rows_bf0,
                     rows_bf1, rows_f0, rows_f1, acc, sem0, sem1, ssem0,
                     ssem1):
    c = lax.axis_index("c")
    s = lax.axis_index("s")
    w = s * NC + c

    z16 = jnp.zeros((16,), jnp.float32)

    def zero_body(t, carry):
        rows_f0[t // 8, pl.ds((t % 8) * 16, 16)] = z16
        return carry

    lax.fori_loop(0, GROUP * 8, zero_body, 0)
    for zi, zn in ((0, 128), (128, 128), (256, 128), (384, 128), (512, 112)):
        pltpu.sync_copy(
            rows_f0.at[pl.ds(0, zn)],
            acc.at[pl.ds(s * ZROWS + zi, zn)],
        )

    @pl.when(s == NS - 1)
    def _zero_tail():
        pltpu.sync_copy(
            rows_f0.at[pl.ds(0, ZTAIL)],
            acc.at[pl.ds(NS * ZROWS, ZTAIL)],
        )

    plsc.subcore_barrier()

    g0 = w * RPW
    sems = (sem0, sem1)

    GR = GROUP // 2
    bufs = (rows_bf0, rows_bf1)
    rows_fs = (rows_f0, rows_f1)
    ssems = (ssem0, ssem1)
    _himask = jnp.full((16,), -65536, jnp.int32)

    def start(g, b):
        sem = sems[b]
        e0 = (g // GPB) * BE + (g % GPB) * 64
        pltpu.async_copy(h_hbm.at[pl.ds(g * GR, GR)], bufs[b], sem)
        pltpu.async_copy(idx_hbm.at[pl.ds(e0, 64)],
                         idx_v.at[b, pl.ds(0, 64)], sem)
        pltpu.async_copy(idx_hbm.at[pl.ds(e0 + HB, 64)],
                         idx_v.at[b, pl.ds(64, 64)], sem)

    def convert(b):
        def cv_body(r, carry):
            for cb in range(EDGE_DIM // 16):
                co = cb * 16
                v = bufs[b][r, pl.ds(co, 16)]
                rows_fs[b][r, pl.ds(co, 16)] = lax.bitcast_convert_type(
                    v << 16, jnp.float32)
                rows_fs[b][GR + r, pl.ds(co, 16)] = lax.bitcast_convert_type(
                    v & _himask, jnp.float32)
            return carry

        lax.fori_loop(0, GR, cv_body, 0)
        for k in range(GROUP // 16):
            idx_sc[b, pl.ds(k * 16, 16)] = idx_v[b, pl.ds(k * 16, 16)]

    def wait_dma(b):
        sem = sems[b]
        pltpu.make_async_copy(
            h_hbm.at[pl.ds(0, GR)], bufs[b], sem).wait()
        pltpu.make_async_copy(
            idx_hbm.at[pl.ds(0, 64)], idx_v.at[b, pl.ds(0, 64)], sem).wait()
        pltpu.make_async_copy(
            idx_hbm.at[pl.ds(0, 64)], idx_v.at[b, pl.ds(64, 64)], sem).wait()

    def start_scatter(b):
        pltpu.async_copy(rows_fs[b], acc.at[idx_sc.at[b]], ssems[b], add=True)

    def wait_scatter(b):
        pltpu.make_async_copy(
            rows_fs[b], acc.at[idx_sc.at[b]], ssems[b]).wait()

    start(g0, 0)

    def group_body(t, carry):
        start(g0 + 2 * t + 1, 1)
        wait_dma(0)

        @pl.when(t > 0)
        def _drain0():
            wait_scatter(0)

        convert(0)
        start_scatter(0)

        @pl.when(2 * t + 2 < RPW)
        def _start_next():
            start(g0 + 2 * t + 2, 0)

        wait_dma(1)

        @pl.when(t > 0)
        def _drain1():
            wait_scatter(1)

        convert(1)
        start_scatter(1)
        return carry

    lax.fori_loop(0, RPW // 2, group_body, 0)
    if RPW % 2 == 1:
        wait_dma(0)
        wait_scatter(0)
        convert(0)
        start_scatter(0)
    wait_scatter(1)
    wait_scatter(0)

    @pl.when(w < N_TAIL)
    def _tail():
        gt = NW * RPW + w
        et = (gt // GPB) * BE + (gt % GPB) * 64
        pltpu.sync_copy(h_hbm.at[pl.ds(gt * GR, GR)], bufs[0])
        pltpu.sync_copy(idx_hbm.at[pl.ds(et, 64)], idx_v.at[0, pl.ds(0, 64)])
        pltpu.sync_copy(idx_hbm.at[pl.ds(et + HB, 64)],
                        idx_v.at[0, pl.ds(64, 64)])
        convert(0)
        pltpu.sync_copy(rows_fs[0], acc.at[idx_sc.at[0]], add=True)

    plsc.subcore_barrier()
    pltpu.sync_copy(
        acc.at[pl.ds(s * ZROWS, ZROWS)],
        out_hbm.at[c, pl.ds(s * ZROWS, ZROWS)],
    )

    @pl.when(s == NS - 1)
    def _write_tail():
        pltpu.sync_copy(
            acc.at[pl.ds(NS * ZROWS, ZTAIL)],
            out_hbm.at[c, pl.ds(NS * ZROWS, ZTAIL)],
        )


def _sc_scatter(h, idx1d):
    mesh = plsc.VectorSubcoreMesh(core_axis_name="c", subcore_axis_name="s")
    kfn = pl.kernel(
        _sc_scatter_body,
        out_type=jax.ShapeDtypeStruct((NC, N_NODES_STATIC, EDGE_DIM), jnp.float32),
        mesh=mesh,
        scratch_types=[
            pltpu.VMEM((2, GROUP), jnp.int32),
            pltpu.VMEM((2, GROUP), jnp.int32),
            pltpu.VMEM((GROUP // 2, EDGE_DIM), jnp.int32),
            pltpu.VMEM((GROUP // 2, EDGE_DIM), jnp.int32),
            pltpu.VMEM((GROUP, EDGE_DIM), jnp.float32),
            pltpu.VMEM((GROUP, EDGE_DIM), jnp.float32),
            pltpu.VMEM_SHARED((N_NODES_STATIC, EDGE_DIM), jnp.float32),
            pltpu.SemaphoreType.DMA,
            pltpu.SemaphoreType.DMA,
            pltpu.SemaphoreType.DMA,
            pltpu.SemaphoreType.DMA,
        ],
    )
    return kfn(h, idx1d)


def _mlp_body(p0_ref, p1_ref, w1, b1, w2, b2, w3, b3, wo, o_ref):
    a = (p0_ref[0] + p0_ref[1]) + (p1_ref[0] + p1_ref[1])
    for wref, bref in ((w1, b1), (w2, b2), (w3, b3)):
        a = jnp.dot(a, wref[...], preferred_element_type=jnp.float32) + bref[...]
        a = a * (1.0 / (1.0 + jnp.exp(-a)))
    o_ref[...] = jnp.dot(a, wo[...], preferred_element_type=jnp.float32)


def _mlp(p0, p1, W1, b1, W2, b2, W3, b3, W_out):
    BN = 2000
    grid = (N_NODES_STATIC // BN,)
    full = lambda shape: pl.BlockSpec(shape, lambda i: tuple(0 for _ in shape))
    return pl.pallas_call(
        _mlp_body,
        grid=grid,
        in_specs=[
            pl.BlockSpec((NC, BN, EDGE_DIM), lambda i: (0, i, 0)),
            pl.BlockSpec((NC, BN, EDGE_DIM), lambda i: (0, i, 0)),
            full((EDGE_DIM, EDGE_DIM)),
            full((1, EDGE_DIM)),
            full((EDGE_DIM, EDGE_DIM)),
            full((1, EDGE_DIM)),
            full((EDGE_DIM, EDGE_DIM)),
            full((1, EDGE_DIM)),
            full((EDGE_DIM, 1)),
        ],
        out_specs=pl.BlockSpec((BN, 1), lambda i: (i, 0)),
        out_shape=jax.ShapeDtypeStruct((N_NODES_STATIC, 1), jnp.float32),
    )(p0, p1, W1, b1.reshape(1, -1), W2, b2.reshape(1, -1),
      W3, b3.reshape(1, -1), W_out)


def kernel(x, rbf, idx_i, num_nodes, W_rbf, W1, b1, W2, b2, W3, b3, W_out):
    idx = idx_i.astype(jnp.int32) + (
        jnp.asarray(num_nodes, jnp.int32) - N_NODES_STATIC)

    partials = []
    for chunk in range(N_CHUNKS):
        h32 = _gating(rbf, x, W_rbf, chunk)
        idx_c = lax.slice(idx, (chunk * CHUNK_EDGES,),
                          ((chunk + 1) * CHUNK_EDGES,))
        partials.append(_sc_scatter(h32, idx_c))
    return _mlp(partials[0], partials[1], W1, b1, W2, b2, W3, b3, W_out)
